# baseline (speedup 1.0000x reference)
import functools

import jax
import jax.numpy as jnp
from jax import lax
from jax.experimental import pallas as pl
from jax.experimental.pallas import tpu as pltpu
from jax.experimental.pallas import tpu_sc as plsc

_EMBED_DIM = 64
_SEQ = 550
_N_IDX = 16384

_NC = 2
_NS = 16
_NW = _NC * _NS
_PER_W = _N_IDX // _NW
_CHUNK = 128
_NCHUNK = _PER_W // _CHUNK
_NLANE = 16
_NSEG = _EMBED_DIM // _NLANE


def _sc_partial_sums(x2d, table):
  mesh = plsc.VectorSubcoreMesh(core_axis_name="c", subcore_axis_name="s")

  @functools.partial(
      pl.kernel,
      out_type=jax.ShapeDtypeStruct((2, _NW, _EMBED_DIM), jnp.float32),
      mesh=mesh,
      scratch_types=[
          pltpu.VMEM((_CHUNK,), jnp.int32),
          pltpu.VMEM((_CHUNK, _EMBED_DIM), jnp.float32),
          pltpu.VMEM((2, _EMBED_DIM), jnp.float32),
          pltpu.SemaphoreType.DMA,
      ],
      compiler_params=pltpu.CompilerParams(use_tc_tiling_on_sc=False),
  )
  def k(x_hbm, table_hbm, out_hbm, idx_v, rows_v, acc_v, sem):
    cid = lax.axis_index("c")
    sid = lax.axis_index("s")
    wid = sid * _NC + cid
    zero = jnp.zeros((_NLANE,), jnp.float32)
    acc0 = (zero,) * _NSEG
    acc1 = (zero,) * _NSEG

    def row_add(j, accs):
      return tuple(
          accs[s] + rows_v[j, pl.ds(s * _NLANE, _NLANE)] for s in range(_NSEG)
      )

    for c in range(_NCHUNK):
      pltpu.sync_copy(x_hbm.at[wid * _NCHUNK + c], idx_v)
      pltpu.async_copy(table_hbm.at[idx_v], rows_v, sem).wait()
      g0 = wid * _PER_W + c * _CHUNK
      n0 = jnp.clip(_SEQ - g0, 0, _CHUNK)
      acc0 = lax.fori_loop(0, n0, row_add, acc0)
      acc1 = lax.fori_loop(n0, _CHUNK, row_add, acc1)

    for s in range(_NSEG):
      acc_v[0, pl.ds(s * _NLANE, _NLANE)] = acc0[s]
      acc_v[1, pl.ds(s * _NLANE, _NLANE)] = acc1[s]
    pltpu.sync_copy(acc_v.at[0], out_hbm.at[0, wid])
    pltpu.sync_copy(acc_v.at[1], out_hbm.at[1, wid])

  return k(x2d, table)


def _tc_finish(partials, fc1_w, fc1_b):
  def body(p_ref, w_ref, b_ref, o_ref):
    p = p_ref[...]
    w = w_ref[...]
    rowdots = jnp.sum(p * w, axis=1, keepdims=True)
    s0 = jnp.sum(rowdots[:_NW]) * (1.0 / _SEQ)
    s1 = jnp.sum(rowdots[_NW:]) * (1.0 / (_N_IDX - _SEQ))
    b = b_ref[0, 0]
    sig0 = 1.0 / (1.0 + jnp.exp(-(s0 + b)))
    sig1 = 1.0 / (1.0 + jnp.exp(-(s1 + b)))
    o_ref[...] = jnp.broadcast_to(0.5 * (sig0 + sig1), (1, 1))

  return pl.pallas_call(
      body,
      out_shape=jax.ShapeDtypeStruct((1, 1), jnp.float32),
  )(partials, fc1_w, fc1_b.reshape(1, 1))


def kernel(x, table, fc1_w, fc1_b):
  x2d = x.astype(jnp.int32).reshape(_NW * _NCHUNK, _CHUNK)
  partials = _sc_partial_sums(x2d, table)
  out = _tc_finish(partials.reshape(2 * _NW, _EMBED_DIM), fc1_w, fc1_b)
  return out[0, 0]


# speedup vs baseline: 1.0142x; 1.0142x over previous
import functools

import jax
import jax.numpy as jnp
from jax import lax
from jax.experimental import pallas as pl
from jax.experimental.pallas import tpu as pltpu
from jax.experimental.pallas import tpu_sc as plsc

_EMBED_DIM = 64
_HALF = _EMBED_DIM // 2
_SEQ = 550
_N_IDX = 16384

_NC = 2
_NS = 16
_NW = _NC * _NS
_PER_W = _N_IDX // _NW
_CHUNK = 128
_NCHUNK = _PER_W // _CHUNK
_NLANE = 16


def _sc_partial_sums(x1d, table_flat):
  mesh = plsc.VectorSubcoreMesh(core_axis_name="c", subcore_axis_name="s")

  @functools.partial(
      pl.kernel,
      out_type=jax.ShapeDtypeStruct((_NW * 2 * _EMBED_DIM,), jnp.float32),
      mesh=mesh,
      scratch_types=[
          pltpu.VMEM((_CHUNK,), jnp.int32),
          pltpu.VMEM((_CHUNK,), jnp.int32),
          pltpu.VMEM((_CHUNK,), jnp.int32),
          pltpu.VMEM((_CHUNK, _HALF), jnp.float32),
          pltpu.VMEM((_CHUNK, _HALF), jnp.float32),
          pltpu.VMEM((2 * _EMBED_DIM,), jnp.float32),
          pltpu.SemaphoreType.DMA,
          pltpu.SemaphoreType.DMA,
      ],
      compiler_params=pltpu.CompilerParams(use_tc_tiling_on_sc=False),
  )
  def k(x_hbm, table2, out_hbm, idx_v, ev_i, od_i, ev_v, od_v, acc_v,
        sem0, sem1):
    cid = lax.axis_index("c")
    sid = lax.axis_index("s")
    wid = sid * _NC + cid
    zero = jnp.zeros((_NLANE,), jnp.float32)
    acc0 = (zero,) * 4
    acc1 = (zero,) * 4

    def row_add(j, accs):
      a0, a1, a2, a3 = accs
      return (
          a0 + ev_v[j, pl.ds(0, _NLANE)],
          a1 + ev_v[j, pl.ds(_NLANE, _NLANE)],
          a2 + od_v[j, pl.ds(0, _NLANE)],
          a3 + od_v[j, pl.ds(_NLANE, _NLANE)],
      )

    for c in range(_NCHUNK):
      base = wid * _PER_W + c * _CHUNK
      pltpu.sync_copy(x_hbm.at[pl.ds(base, _CHUNK)], idx_v)
      for q in range(_CHUNK // _NLANE):
        sl = pl.ds(q * _NLANE, _NLANE)
        v = idx_v[sl]
        ev_i[sl] = v + v
        od_i[sl] = v + v + 1
      cp0 = pltpu.async_copy(table2.at[ev_i], ev_v, sem0)
      cp1 = pltpu.async_copy(table2.at[od_i], od_v, sem1)
      cp0.wait()
      cp1.wait()
      n0 = jnp.clip(_SEQ - base, 0, _CHUNK)
      acc0 = lax.fori_loop(0, n0, row_add, acc0)
      acc1 = lax.fori_loop(n0, _CHUNK, row_add, acc1)

    for s in range(4):
      acc_v[pl.ds(s * _NLANE, _NLANE)] = acc0[s]
      acc_v[pl.ds(_EMBED_DIM + s * _NLANE, _NLANE)] = acc1[s]
    pltpu.sync_copy(acc_v, out_hbm.at[pl.ds(wid * 2 * _EMBED_DIM,
                                            2 * _EMBED_DIM)])

  return k(x1d, table_flat)


def _tc_finish(partials, fc1_w, fc1_b):
  def body(p_ref, w_ref, b_ref, o_ref):
    p = p_ref[...].reshape(_NW, 2 * _EMBED_DIM)
    s = jnp.sum(p, axis=0, keepdims=True)
    w = w_ref[...]
    d0 = jnp.sum(s[:, :_EMBED_DIM] * w) * (1.0 / _SEQ)
    d1 = jnp.sum(s[:, _EMBED_DIM:] * w) * (1.0 / (_N_IDX - _SEQ))
    b = b_ref[0, 0]
    sig0 = 1.0 / (1.0 + jnp.exp(-(d0 + b)))
    sig1 = 1.0 / (1.0 + jnp.exp(-(d1 + b)))
    o_ref[...] = jnp.broadcast_to(0.5 * (sig0 + sig1), (1, 1))

  return pl.pallas_call(
      body,
      out_shape=jax.ShapeDtypeStruct((1, 1), jnp.float32),
  )(partials, fc1_w, fc1_b.reshape(1, 1))


def kernel(x, table, fc1_w, fc1_b):
  x1d = x.astype(jnp.int32).reshape(_N_IDX)
  table_half = table.reshape(2 * table.shape[0], _HALF)
  partials = _sc_partial_sums(x1d, table_half)
  out = _tc_finish(partials, fc1_w, fc1_b)
  return out[0, 0]


# speedup vs baseline: 1.4868x; 1.4660x over previous
import functools

import jax
import jax.numpy as jnp
from jax import lax
from jax.experimental import pallas as pl
from jax.experimental.pallas import tpu as pltpu
from jax.experimental.pallas import tpu_sc as plsc

_EMBED_DIM = 64
_SEQ = 550
_N_IDX = 16384

_NC = 2
_NS = 16
_NW = _NC * _NS
_PER_W = _N_IDX // _NW
_K = 64
_NCHUNK = _PER_W // _K
_NLANE = 16
_NSEG = _EMBED_DIM // _NLANE


def _sc_partial_sums(x1d, table):
  mesh = plsc.VectorSubcoreMesh(core_axis_name="c", subcore_axis_name="s")

  @functools.partial(
      pl.kernel,
      out_type=jax.ShapeDtypeStruct((_NW * 2 * _EMBED_DIM,), jnp.float32),
      mesh=mesh,
      scratch_types=[
          pltpu.VMEM((_PER_W,), jnp.int32),
          pltpu.VMEM((2, _K, _EMBED_DIM), jnp.float32),
          pltpu.VMEM((2 * _EMBED_DIM,), jnp.float32),
          pltpu.SemaphoreType.DMA,
          pltpu.SemaphoreType.DMA,
          pltpu.SemaphoreType.DMA,
      ],
  )
  def k(x_hbm, table_hbm, out_hbm, idx_v, rows_v, acc_v, sem0, sem1, sem_i):
    cid = lax.axis_index("c")
    sid = lax.axis_index("s")
    wid = sid * _NC + cid
    base = wid * _PER_W
    pltpu.async_copy(x_hbm.at[pl.ds(base, _PER_W)], idx_v, sem_i).wait()

    sems = (sem0, sem1)

    def enqueue(c, buf):
      @pl.loop(0, _K // _NLANE)
      def _(g):
        v = idx_v[pl.ds(c * _K + g * _NLANE, _NLANE)]
        for l in range(_NLANE):
          pltpu.async_copy(
              table_hbm.at[v[l]], rows_v.at[buf, g * _NLANE + l], sems[buf]
          )

    def drain(buf):
      pltpu.make_async_copy(
          table_hbm.at[pl.ds(0, _K)], rows_v.at[buf], sems[buf]
      ).wait()

    def row_adder(buf):
      def row_add(j, accs):
        return tuple(
            accs[s] + rows_v[buf, j, pl.ds(s * _NLANE, _NLANE)]
            for s in range(_NSEG)
        )
      return row_add

    zero = jnp.zeros((_NLANE,), jnp.float32)
    acc0 = (zero,) * _NSEG
    acc1 = (zero,) * _NSEG

    enqueue(0, 0)
    for c in range(_NCHUNK):
      buf = c & 1
      if c + 1 < _NCHUNK:
        enqueue(c + 1, (c + 1) & 1)
      drain(buf)
      g = base + c * _K
      n0 = jnp.clip(_SEQ - g, 0, _K)
      acc0 = lax.fori_loop(0, n0, row_adder(buf), acc0)
      acc1 = lax.fori_loop(n0, _K, row_adder(buf), acc1)

    for s in range(_NSEG):
      acc_v[pl.ds(s * _NLANE, _NLANE)] = acc0[s]
      acc_v[pl.ds(_EMBED_DIM + s * _NLANE, _NLANE)] = acc1[s]
    pltpu.sync_copy(acc_v, out_hbm.at[pl.ds(wid * 2 * _EMBED_DIM,
                                            2 * _EMBED_DIM)])

  return k(x1d, table)


def _tc_finish(partials, fc1_w, fc1_b):
  def body(p_ref, w_ref, b_ref, o_ref):
    p = p_ref[...].reshape(_NW, 2 * _EMBED_DIM)
    s = jnp.sum(p, axis=0, keepdims=True)
    w = w_ref[...]
    d0 = jnp.sum(s[:, :_EMBED_DIM] * w) * (1.0 / _SEQ)
    d1 = jnp.sum(s[:, _EMBED_DIM:] * w) * (1.0 / (_N_IDX - _SEQ))
    b = b_ref[0, 0]
    sig0 = 1.0 / (1.0 + jnp.exp(-(d0 + b)))
    sig1 = 1.0 / (1.0 + jnp.exp(-(d1 + b)))
    o_ref[...] = jnp.broadcast_to(0.5 * (sig0 + sig1), (1, 1))

  return pl.pallas_call(
      body,
      out_shape=jax.ShapeDtypeStruct((1, 1), jnp.float32),
  )(partials, fc1_w, fc1_b.reshape(1, 1))


def kernel(x, table, fc1_w, fc1_b):
  x1d = x.astype(jnp.int32).reshape(_N_IDX)
  partials = _sc_partial_sums(x1d, table)
  out = _tc_finish(partials, fc1_w, fc1_b)
  return out[0, 0]
